# baseline (speedup 1.0000x reference)
import jax
import jax.numpy as jnp
from jax import lax
from jax.experimental import pallas as pl
from jax.experimental.pallas import tpu as pltpu
from jax.experimental.pallas import tpu_sc as plsc

_VOCAB = 1000000
_DIM = 64
_SEQ = 200
_BATCH = 4096

_NC, _NS = 2, 16
_NW = _NC * _NS
_ROWS = _BATCH * _SEQ
_RPW = _ROWS // _NW
_CH = 128
_NCHUNK = _RPW // _CH


def _body(x_hbm, table_hbm, pos_hbm, out_hbm, idx_v, buf_v, pos2_sh, sem):
    sid = lax.axis_index("s")
    wid = sid * _NC + lax.axis_index("c")

    @pl.when(sid == 0)
    def _fill():
        pltpu.sync_copy(pos_hbm, pos2_sh.at[pl.ds(0, _SEQ)])
        pltpu.sync_copy(pos_hbm, pos2_sh.at[pl.ds(_SEQ, _SEQ)])

    plsc.subcore_barrier()

    @pl.loop(0, _NCHUNK)
    def _chunk(k):
        base = wid * _RPW + k * _CH
        off = lax.rem(k * _CH, _SEQ)
        pltpu.sync_copy(x_hbm.at[pl.ds(base, _CH)], idx_v)
        pltpu.sync_copy(pos2_sh.at[pl.ds(off, _CH)], buf_v)
        pltpu.async_copy(table_hbm.at[idx_v], buf_v, sem, add=True).wait()
        pltpu.sync_copy(buf_v, out_hbm.at[pl.ds(base, _CH)])


def kernel(x, table, pos):
    xf = x.reshape(_ROWS)
    run = pl.kernel(
        _body,
        out_type=jax.ShapeDtypeStruct((_ROWS, _DIM), jnp.float32),
        mesh=plsc.VectorSubcoreMesh(core_axis_name="c", subcore_axis_name="s"),
        scratch_types=[
            pltpu.VMEM((_CH,), jnp.int32),
            pltpu.VMEM((_CH, _DIM), jnp.float32),
            pltpu.VMEM_SHARED((2 * _SEQ, _DIM), jnp.float32),
            pltpu.SemaphoreType.DMA,
        ],
        compiler_params=pltpu.CompilerParams(use_tc_tiling_on_sc=False),
    )
    out = run(xf, table, pos)
    return out.reshape(_BATCH, _SEQ, _DIM)


# speedup vs baseline: 1.0937x; 1.0937x over previous
import jax
import jax.numpy as jnp
from jax import lax
from jax.experimental import pallas as pl
from jax.experimental.pallas import tpu as pltpu
from jax.experimental.pallas import tpu_sc as plsc

_VOCAB = 1000000
_DIM = 64
_SEQ = 200
_BATCH = 4096

_NC, _NS = 2, 16
_NW = _NC * _NS
_ROWS = _BATCH * _SEQ
_RPW = _ROWS // _NW
_CH = 256
_NSUB = _CH // 128
_NCHUNK = _RPW // _CH
_POSREP = 4


def _body(x_hbm, table_hbm, pos_hbm, out_hbm, idx_v, buf_v, pos2_sh, sem):
    sid = lax.axis_index("s")
    wid = sid * _NC + lax.axis_index("c")

    @pl.when(sid == 0)
    def _fill_pos():
        for r in range(_POSREP):
            pltpu.sync_copy(pos_hbm, pos2_sh.at[pl.ds(r * _SEQ, _SEQ)])

    plsc.subcore_barrier()

    @pl.loop(0, _NCHUNK)
    def _chunk(k):
        base = wid * _RPW + k * _CH
        off = lax.rem(k * _CH, _SEQ)
        pltpu.sync_copy(x_hbm.at[pl.ds(base, _CH)], idx_v)
        pltpu.sync_copy(pos2_sh.at[pl.ds(off, _CH)], buf_v)
        descs = [
            pltpu.async_copy(
                table_hbm.at[idx_v.at[pl.ds(j * 128, 128)]],
                buf_v.at[pl.ds(j * 128, 128)],
                sem,
                add=True,
            )
            for j in range(_NSUB)
        ]
        for d in descs:
            d.wait()
        pltpu.sync_copy(buf_v, out_hbm.at[pl.ds(base, _CH)])


def kernel(x, table, pos):
    xf = x.reshape(_ROWS)
    run = pl.kernel(
        _body,
        out_type=jax.ShapeDtypeStruct((_ROWS, _DIM), jnp.float32),
        mesh=plsc.VectorSubcoreMesh(core_axis_name="c", subcore_axis_name="s"),
        scratch_types=[
            pltpu.VMEM((_CH,), jnp.int32),
            pltpu.VMEM((_CH, _DIM), jnp.float32),
            pltpu.VMEM_SHARED((_POSREP * _SEQ, _DIM), jnp.float32),
            pltpu.SemaphoreType.DMA,
        ],
        compiler_params=pltpu.CompilerParams(use_tc_tiling_on_sc=False),
    )
    out = run(xf, table, pos)
    return out.reshape(_BATCH, _SEQ, _DIM)


# speedup vs baseline: 1.1403x; 1.0426x over previous
import jax
import jax.numpy as jnp
from jax import lax
from jax.experimental import pallas as pl
from jax.experimental.pallas import tpu as pltpu
from jax.experimental.pallas import tpu_sc as plsc

_VOCAB = 1000000
_DIM = 64
_SEQ = 200
_BATCH = 4096

_NC, _NS = 2, 16
_NW = _NC * _NS
_ROWS = _BATCH * _SEQ
_RPW = _ROWS // _NW
_CH = 256
_NSUB = _CH // 128
_NB = 2
_NCHUNK = _RPW // _CH
_POSREP = 4


def _body(x_hbm, table_hbm, pos_hbm, out_hbm, idx_v, buf_v, pos2_sh, g0, g1, w0, w1):
    gsems, wsems = [g0, g1], [w0, w1]
    sid = lax.axis_index("s")
    wid = sid * _NC + lax.axis_index("c")

    @pl.when(sid == 0)
    def _fill_pos():
        for r in range(_POSREP):
            pltpu.sync_copy(pos_hbm, pos2_sh.at[pl.ds(r * _SEQ, _SEQ)])

    plsc.subcore_barrier()

    def wait_write(b):
        pltpu.make_async_copy(buf_v.at[b], out_hbm.at[pl.ds(0, _CH)], wsems[b]).wait()

    def step(k, b, recycle):
        base = wid * _RPW + k * _CH
        off = lax.rem(k * _CH, _SEQ)
        pltpu.sync_copy(x_hbm.at[pl.ds(base, _CH)], idx_v.at[b])
        if recycle:
            wait_write(b)
        pltpu.sync_copy(pos2_sh.at[pl.ds(off, _CH)], buf_v.at[b])
        descs = [
            pltpu.async_copy(
                table_hbm.at[idx_v.at[b, pl.ds(j * 128, 128)]],
                buf_v.at[b, pl.ds(j * 128, 128)],
                gsems[b],
                add=True,
            )
            for j in range(_NSUB)
        ]
        for d in descs:
            d.wait()
        pltpu.async_copy(buf_v.at[b], out_hbm.at[pl.ds(base, _CH)], wsems[b])

    for db in range(_NB):
        step(db, db, recycle=False)

    @pl.loop(1, _NCHUNK // _NB)
    def _grp(g):
        for db in range(_NB):
            step(g * _NB + db, db, recycle=True)

    for db in range(_NB):
        wait_write(db)


def kernel(x, table, pos):
    xf = x.reshape(_ROWS)
    run = pl.kernel(
        _body,
        out_type=jax.ShapeDtypeStruct((_ROWS, _DIM), jnp.float32),
        mesh=plsc.VectorSubcoreMesh(core_axis_name="c", subcore_axis_name="s"),
        scratch_types=[
            pltpu.VMEM((_NB, _CH), jnp.int32),
            pltpu.VMEM((_NB, _CH, _DIM), jnp.float32),
            pltpu.VMEM_SHARED((_POSREP * _SEQ, _DIM), jnp.float32),
        ] + [pltpu.SemaphoreType.DMA] * (2 * _NB),
        compiler_params=pltpu.CompilerParams(use_tc_tiling_on_sc=False),
    )
    out = run(xf, table, pos)
    return out.reshape(_BATCH, _SEQ, _DIM)
